# baseline (speedup 1.0000x reference)
import functools

import jax
import jax.numpy as jnp
from jax import lax
from jax.experimental import pallas as pl
from jax.experimental.pallas import tpu as pltpu
from jax.experimental.pallas import tpu_sc as plsc

F = 32
POP = 4
CON = 4
ENV = POP + CON
B = 16384

NC = 2
NS = 16
NW = NC * NS
BPW = B // NW
KCH = 128
NK = BPW // KCH
L = 16
NG = BPW // L


def _sc_body(user_t, item_t, uenv_t, ienv_t, pop_t, con_t, w_hbm, b_hbm,
             uid_hbm, iid_hbm, pid_hbm, cid_hbm,
             mf_out, score_out, env_out,
             uid_v, iid_v, pid_v, cid_v,
             u_rows, it_rows, ue_rows, ie_rows,
             pop_v, con_v, w_v, b_v,
             mf_buf, s_buf, o_buf, sem):
    wid = lax.axis_index("s") * NC + lax.axis_index("c")
    base = wid * BPW

    for k in range(NK):
        pltpu.sync_copy(uid_hbm.at[pl.ds(base + k * KCH, KCH)], uid_v.at[k])
        pltpu.sync_copy(iid_hbm.at[pl.ds(base + k * KCH, KCH)], iid_v.at[k])
    pltpu.sync_copy(pid_hbm.at[pl.ds(base, BPW)], pid_v)
    pltpu.sync_copy(cid_hbm.at[pl.ds(base, BPW)], cid_v)
    pltpu.sync_copy(pop_t, pop_v)
    pltpu.sync_copy(con_t, con_v)
    pltpu.sync_copy(w_hbm, w_v)
    pltpu.sync_copy(b_hbm, b_v)

    copies = []
    for k in range(NK):
        sl = pl.ds(k * KCH, KCH)
        copies.append(pltpu.async_copy(user_t.at[uid_v.at[k]], u_rows.at[sl], sem))
        copies.append(pltpu.async_copy(item_t.at[iid_v.at[k]], it_rows.at[sl], sem))
        copies.append(pltpu.async_copy(uenv_t.at[uid_v.at[k]], ue_rows.at[sl], sem))
        copies.append(pltpu.async_copy(ienv_t.at[iid_v.at[k]], ie_rows.at[sl], sem))
    for cp in copies:
        cp.wait()

    lane = lax.iota(jnp.int32, L)

    def group(g, _):
        row_idx = g * L + lane
        pid = pid_v[pl.ds(g * L, L)]
        cid = cid_v[pl.ds(g * L, L)]
        mf_acc = jnp.zeros((L,), jnp.float32)
        s_acc = jnp.zeros((L,), jnp.float32)
        o_acc = [jnp.zeros((L,), jnp.float32) for _ in range(ENV)]
        for f in range(F):
            fidx = jnp.full((L,), f, jnp.int32)
            pv = plsc.load_gather(pop_v, [pid, fidx])
            cv = plsc.load_gather(con_v, [cid, fidx])
            env = pv * cv
            u = plsc.load_gather(u_rows, [row_idx, fidx])
            it = plsc.load_gather(it_rows, [row_idx, fidx])
            ue = plsc.load_gather(ue_rows, [row_idx, fidx])
            ie = plsc.load_gather(ie_rows, [row_idx, fidx])
            mf_acc = mf_acc + (u * it) * env
            pref = (ue * ie) * env
            s_acc = s_acc + pref
            for j in range(ENV):
                o_acc[j] = o_acc[j] + pref * w_v[j * F + f, :]
        zero = jnp.zeros((L,), jnp.float32)
        plsc.store_scatter(mf_buf, [row_idx], jnp.maximum(mf_acc, zero))
        plsc.store_scatter(s_buf, [row_idx], jnp.maximum(s_acc, zero))
        for j in range(ENV):
            jidx = jnp.full((L,), j, jnp.int32)
            plsc.store_scatter(o_buf, [row_idx, jidx], o_acc[j] + b_v[j, :])
        return _

    lax.fori_loop(0, NG, group, None)

    pltpu.sync_copy(mf_buf, mf_out.at[pl.ds(base, BPW)])
    pltpu.sync_copy(s_buf, score_out.at[pl.ds(base, BPW)])
    pltpu.sync_copy(o_buf, env_out.at[pl.ds(base, BPW)])


@jax.jit
def _run(user_t, item_t, uenv_t, ienv_t, pop_t, con_t, W, b16,
         uid, iid, pid, cid):
    mesh = plsc.VectorSubcoreMesh(core_axis_name="c", subcore_axis_name="s")
    f32 = jnp.float32
    return pl.kernel(
        _sc_body,
        out_type=[
            jax.ShapeDtypeStruct((B,), f32),
            jax.ShapeDtypeStruct((B,), f32),
            jax.ShapeDtypeStruct((B, ENV), f32),
        ],
        mesh=mesh,
        compiler_params=pltpu.CompilerParams(
            needs_layout_passes=False, use_tc_tiling_on_sc=False),
        scratch_types=[
            pltpu.VMEM((NK, KCH), jnp.int32),
            pltpu.VMEM((NK, KCH), jnp.int32),
            pltpu.VMEM((BPW,), jnp.int32),
            pltpu.VMEM((BPW,), jnp.int32),
            pltpu.VMEM((BPW, F), f32),
            pltpu.VMEM((BPW, F), f32),
            pltpu.VMEM((BPW, F), f32),
            pltpu.VMEM((BPW, F), f32),
            pltpu.VMEM((POP, F), f32),
            pltpu.VMEM((CON, F), f32),
            pltpu.VMEM((ENV * F, L), f32),
            pltpu.VMEM((ENV, L), f32),
            pltpu.VMEM((BPW,), f32),
            pltpu.VMEM((BPW,), f32),
            pltpu.VMEM((BPW, ENV), f32),
            pltpu.SemaphoreType.DMA,
        ],
    )(user_t, item_t, uenv_t, ienv_t, pop_t, con_t, W, b16,
      uid, iid, pid, cid)


def kernel(user_table, item_table, user_env_table, item_env_table,
           pop_table, con_table, W, b,
           users_id, items_id, pop_envs_id, con_envs_id):
    uid = users_id.astype(jnp.int32)
    iid = items_id.astype(jnp.int32)
    pid = pop_envs_id.astype(jnp.int32)
    cid = con_envs_id.astype(jnp.int32)
    w_bc = jnp.broadcast_to(W.reshape(ENV * F, 1), (ENV * F, L)) + 0.0
    b_bc = jnp.broadcast_to(b.reshape(ENV, 1), (ENV, L)) + 0.0
    mf, s, o = _run(user_table, item_table, user_env_table, item_env_table,
                    pop_table, con_table, w_bc, b_bc, uid, iid, pid, cid)
    return (mf, s, o)
